# baseline (speedup 1.0000x reference)
import functools

import jax
import jax.numpy as jnp
from jax import lax
from jax.experimental import pallas as pl
from jax.experimental.pallas import tpu as pltpu
from jax.experimental.pallas import tpu_sc as plsc

B = 16
C = 3
H = W = 256
P = 16
NH = H // P
NW_ = W // P
N = NH * NW_
PATCH_DIM = C * P * P
LATENT = 256
K = 8192
BETA = 0.25
ROWS = B * N

HALF_B = B // 2
HALF_ROWS = HALF_B * N
ROW_TILE = 256
CODE_CHUNK = 512
N_CHUNKS = K // CODE_CHUNK


def _enc_vq_body(x_ref, w_ref, cb_hbm, z_ref, idx_ref, cb_vmem, cn_ref, sems):
    i = pl.program_id(0)

    @pl.when(i == 0)
    def _():
        for c in range(N_CHUNKS):
            pltpu.make_async_copy(
                cb_hbm.at[pl.ds(c * CODE_CHUNK, CODE_CHUNK), :],
                cb_vmem.at[pl.ds(c * CODE_CHUNK, CODE_CHUNK), :],
                sems.at[c],
            ).start()

    xb = x_ref[0]
    x5 = xb.reshape(C, NH, P, NW_, P)
    pf = x5.transpose(1, 3, 0, 2, 4).reshape(N, PATCH_DIM)

    z = jnp.dot(pf, w_ref[...], preferred_element_type=jnp.float32)
    z_ref[...] = z
    zn = jnp.sum(z * z, axis=1, keepdims=True)
    best_v = None
    best_i = None
    for c in range(N_CHUNKS):
        @pl.when(i == 0)
        def _(c=c):
            pltpu.make_async_copy(
                cb_hbm.at[pl.ds(c * CODE_CHUNK, CODE_CHUNK), :],
                cb_vmem.at[pl.ds(c * CODE_CHUNK, CODE_CHUNK), :],
                sems.at[c],
            ).wait()
            cbw = cb_vmem[c * CODE_CHUNK:(c + 1) * CODE_CHUNK, :]
            cn_ref[0, c * CODE_CHUNK:(c + 1) * CODE_CHUNK] = jnp.sum(cbw * cbw, axis=1)

        cb = cb_vmem[c * CODE_CHUNK:(c + 1) * CODE_CHUNK, :]
        cn = cn_ref[:, c * CODE_CHUNK:(c + 1) * CODE_CHUNK]
        mm = lax.dot_general(z, cb, (((1,), (1,)), ((), ())),
                             preferred_element_type=jnp.float32)
        d = (zn + cn) - 2.0 * mm
        vmin = jnp.min(d, axis=1, keepdims=True)
        col = lax.broadcasted_iota(jnp.int32, d.shape, 1) + (c * CODE_CHUNK)
        imin = jnp.min(jnp.where(d == vmin, col, jnp.int32(K)),
                       axis=1, keepdims=True)
        if best_v is None:
            best_v, best_i = vmin, imin
        else:
            take_new = vmin < best_v
            best_i = jnp.where(take_new, imin, best_i)
            best_v = jnp.where(take_new, vmin, best_v)
    idx_ref[...] = best_i[:, 0].reshape(1, 1, ROW_TILE)


def _encode_vq(x, enc_W, codebook, half):
    base = half * HALF_B
    return pl.pallas_call(
        _enc_vq_body,
        grid=(HALF_B,),
        in_specs=[
            pl.BlockSpec((1, C, H, W), lambda i, b=base: (i + b, 0, 0, 0)),
            pl.BlockSpec((PATCH_DIM, LATENT), lambda i: (0, 0)),
            pl.BlockSpec(memory_space=pltpu.MemorySpace.HBM),
        ],
        out_specs=[
            pl.BlockSpec((ROW_TILE, LATENT), lambda i: (i, 0)),
            pl.BlockSpec((1, 1, ROW_TILE), lambda i: (i, 0, 0)),
        ],
        out_shape=[
            jax.ShapeDtypeStruct((HALF_ROWS, LATENT), jnp.float32),
            jax.ShapeDtypeStruct((HALF_B, 1, ROW_TILE), jnp.int32),
        ],
        scratch_shapes=[
            pltpu.VMEM((K, LATENT), jnp.float32),
            pltpu.VMEM((1, K), jnp.float32),
            pltpu.SemaphoreType.DMA((N_CHUNKS,)),
        ],
    )(x, enc_W, codebook)


def _dec_body(z_ref, zq_ref, w_ref, prev_ref, out_ref, loss_ref):
    del prev_ref
    i = pl.program_id(0)
    z = z_ref[...]
    zq = zq_ref[...]
    zq_st = z + (zq - z)
    dec = jnp.dot(zq_st, w_ref[...], preferred_element_type=jnp.float32)
    d5 = dec.reshape(NH, NW_, C, P, P)
    out_ref[...] = d5.transpose(2, 0, 3, 1, 4).reshape(1, C, H, W)
    dz = z - zq
    part = jnp.sum(dz * dz).reshape(1, 1)

    @pl.when(i == 0)
    def _():
        loss_ref[...] = part

    @pl.when(i > 0)
    def _():
        loss_ref[...] = loss_ref[...] + part


def _decode(z, zq, dec_W, prev, half):
    base = half * HALF_B
    in_specs = [
        pl.BlockSpec((ROW_TILE, LATENT), lambda i: (i, 0)),
        pl.BlockSpec((ROW_TILE, LATENT), lambda i: (i, 0)),
        pl.BlockSpec((LATENT, PATCH_DIM), lambda i: (0, 0)),
    ]
    args = [z, zq, dec_W]
    aliases = {}
    body = _dec_body
    if prev is not None:
        in_specs.append(pl.BlockSpec(memory_space=pltpu.MemorySpace.HBM))
        args.append(prev)
        aliases = {3: 0}
    else:
        def body(z_ref, zq_ref, w_ref, out_ref, loss_ref):
            return _dec_body(z_ref, zq_ref, w_ref, None, out_ref, loss_ref)
    return pl.pallas_call(
        body,
        grid=(HALF_B,),
        in_specs=in_specs,
        out_specs=[
            pl.BlockSpec((1, C, H, W), lambda i, b=base: (i + b, 0, 0, 0)),
            pl.BlockSpec((1, 1), lambda i: (0, 0)),
        ],
        out_shape=[
            jax.ShapeDtypeStruct((B, C, H, W), jnp.float32),
            jax.ShapeDtypeStruct((1, 1), jnp.float32),
        ],
        input_output_aliases=aliases,
    )(*args)


def _sc_gather(codebook, idx):
    nrows = idx.shape[0]
    info = plsc.get_sparse_core_info()
    nc, ns = info.num_cores, info.num_subcores
    nworkers = nc * ns
    bpw = nrows // nworkers
    hw = bpw // 2
    mesh = plsc.VectorSubcoreMesh(core_axis_name="c", subcore_axis_name="s")

    @functools.partial(
        pl.kernel,
        mesh=mesh,
        out_type=jax.ShapeDtypeStruct((nrows, LATENT), jnp.float32),
        scratch_types=[
            pltpu.VMEM((hw,), jnp.int32),
            pltpu.VMEM((hw,), jnp.int32),
            pltpu.VMEM((hw, LATENT), jnp.float32),
            pltpu.VMEM((hw, LATENT), jnp.float32),
            pltpu.SemaphoreType.DMA,
            pltpu.SemaphoreType.DMA,
        ],
    )
    def gather_kernel(cb_hbm, idx_hbm, out_hbm, idx_a, idx_b, rows_a, rows_b,
                      sem_a, sem_b):
        wid = lax.axis_index("s") * nc + lax.axis_index("c")
        base = wid * bpw
        pltpu.sync_copy(idx_hbm.at[pl.ds(base, hw)], idx_a)
        pltpu.sync_copy(idx_hbm.at[pl.ds(base + hw, hw)], idx_b)
        ca = pltpu.async_copy(cb_hbm.at[idx_a], rows_a, sem_a)
        cb = pltpu.async_copy(cb_hbm.at[idx_b], rows_b, sem_b)
        ca.wait()
        pltpu.sync_copy(rows_a, out_hbm.at[pl.ds(base, hw)])
        cb.wait()
        pltpu.sync_copy(rows_b, out_hbm.at[pl.ds(base + hw, hw)])

    return gather_kernel(codebook, idx)


def kernel(x, enc_W, dec_W, codebook):
    z0, i0 = _encode_vq(x, enc_W, codebook, 0)
    z1, i1 = _encode_vq(x, enc_W, codebook, 1)
    zq0 = _sc_gather(codebook, i0.reshape(HALF_ROWS))
    zq1 = _sc_gather(codebook, i1.reshape(HALF_ROWS))
    d0, l0 = _decode(z0, zq0, dec_W, None, 0)
    decoded_images, l1 = _decode(z1, zq1, dec_W, d0, 1)
    codebook_indices = jnp.concatenate(
        [i0.reshape(HALF_B, N), i1.reshape(HALF_B, N)], axis=0)
    m = (l0[0, 0] + l1[0, 0]) / jnp.float32(ROWS * LATENT)
    q_loss = m + jnp.float32(BETA) * m
    return decoded_images, codebook_indices, q_loss


# speedup vs baseline: 1.3189x; 1.3189x over previous
import functools

import jax
import jax.numpy as jnp
from jax import lax
from jax.experimental import pallas as pl
from jax.experimental.pallas import tpu as pltpu
from jax.experimental.pallas import tpu_sc as plsc

B = 16
C = 3
H = W = 256
P = 16
NH = H // P
NW_ = W // P
N = NH * NW_
PATCH_DIM = C * P * P
LATENT = 256
K = 8192
BETA = 0.25
ROWS = B * N

HALF_B = B // 2
HALF_ROWS = HALF_B * N
ROW_TILE = 256
CODE_CHUNK = 512
N_CHUNKS = K // CODE_CHUNK


def _enc_vq_body(x_ref, w_ref, cb_ref, z_ref, idx_ref, cn_ref):
    i = pl.program_id(0)

    @pl.when(i == 0)
    def _():
        for c in range(N_CHUNKS):
            cb = cb_ref[c * CODE_CHUNK:(c + 1) * CODE_CHUNK, :]
            cn_ref[0, c * CODE_CHUNK:(c + 1) * CODE_CHUNK] = jnp.sum(cb * cb, axis=1)

    xb = x_ref[0]
    x5 = xb.reshape(C, NH, P, NW_, P)
    pf = x5.transpose(1, 3, 0, 2, 4).reshape(N, PATCH_DIM)

    z = jnp.dot(pf, w_ref[...], preferred_element_type=jnp.float32)
    z_ref[...] = z
    zn = jnp.sum(z * z, axis=1, keepdims=True)
    best_v = None
    best_i = None
    for c in range(N_CHUNKS):
        cb = cb_ref[c * CODE_CHUNK:(c + 1) * CODE_CHUNK, :]
        cn = cn_ref[:, c * CODE_CHUNK:(c + 1) * CODE_CHUNK]
        mm = lax.dot_general(z, cb, (((1,), (1,)), ((), ())),
                             preferred_element_type=jnp.float32)
        d = (zn + cn) - 2.0 * mm
        vmin = jnp.min(d, axis=1, keepdims=True)
        col = lax.broadcasted_iota(jnp.int32, d.shape, 1) + (c * CODE_CHUNK)
        imin = jnp.min(jnp.where(d == vmin, col, jnp.int32(K)),
                       axis=1, keepdims=True)
        if best_v is None:
            best_v, best_i = vmin, imin
        else:
            take_new = vmin < best_v
            best_i = jnp.where(take_new, imin, best_i)
            best_v = jnp.where(take_new, vmin, best_v)
    idx_ref[...] = best_i[:, 0].reshape(1, 1, ROW_TILE)


def _encode_vq(x, enc_W, codebook, half):
    base = half * HALF_B
    return pl.pallas_call(
        _enc_vq_body,
        grid=(HALF_B,),
        in_specs=[
            pl.BlockSpec((1, C, H, W), lambda i, b=base: (i + b, 0, 0, 0)),
            pl.BlockSpec((PATCH_DIM, LATENT), lambda i: (0, 0)),
            pl.BlockSpec((K, LATENT), lambda i: (0, 0)),
        ],
        out_specs=[
            pl.BlockSpec((ROW_TILE, LATENT), lambda i: (i, 0)),
            pl.BlockSpec((1, 1, ROW_TILE), lambda i: (i, 0, 0)),
        ],
        out_shape=[
            jax.ShapeDtypeStruct((HALF_ROWS, LATENT), jnp.float32),
            jax.ShapeDtypeStruct((HALF_B, 1, ROW_TILE), jnp.int32),
        ],
        scratch_shapes=[pltpu.VMEM((1, K), jnp.float32)],
    )(x, enc_W, codebook)


def _dec_body(z_ref, zq_ref, w_ref, prev_ref, out_ref, loss_ref):
    del prev_ref
    i = pl.program_id(0)
    z = z_ref[...]
    zq = zq_ref[...]
    zq_st = z + (zq - z)
    dec = jnp.dot(zq_st, w_ref[...], preferred_element_type=jnp.float32)
    d5 = dec.reshape(NH, NW_, C, P, P)
    out_ref[...] = d5.transpose(2, 0, 3, 1, 4).reshape(1, C, H, W)
    dz = z - zq
    part = jnp.sum(dz * dz).reshape(1, 1)

    @pl.when(i == 0)
    def _():
        loss_ref[...] = part

    @pl.when(i > 0)
    def _():
        loss_ref[...] = loss_ref[...] + part


def _decode(z, zq, dec_W, prev, half):
    base = half * HALF_B
    in_specs = [
        pl.BlockSpec((ROW_TILE, LATENT), lambda i: (i, 0)),
        pl.BlockSpec((ROW_TILE, LATENT), lambda i: (i, 0)),
        pl.BlockSpec((LATENT, PATCH_DIM), lambda i: (0, 0)),
    ]
    args = [z, zq, dec_W]
    aliases = {}
    body = _dec_body
    if prev is not None:
        in_specs.append(pl.BlockSpec(memory_space=pltpu.MemorySpace.HBM))
        args.append(prev)
        aliases = {3: 0}
    else:
        def body(z_ref, zq_ref, w_ref, out_ref, loss_ref):
            return _dec_body(z_ref, zq_ref, w_ref, None, out_ref, loss_ref)
    return pl.pallas_call(
        body,
        grid=(HALF_B,),
        in_specs=in_specs,
        out_specs=[
            pl.BlockSpec((1, C, H, W), lambda i, b=base: (i + b, 0, 0, 0)),
            pl.BlockSpec((1, 1), lambda i: (0, 0)),
        ],
        out_shape=[
            jax.ShapeDtypeStruct((B, C, H, W), jnp.float32),
            jax.ShapeDtypeStruct((1, 1), jnp.float32),
        ],
        input_output_aliases=aliases,
    )(*args)


def _sc_gather(codebook, idx):
    nrows = idx.shape[0]
    info = plsc.get_sparse_core_info()
    nc, ns = info.num_cores, info.num_subcores
    nworkers = nc * ns
    bpw = nrows // nworkers
    hw = bpw // 2
    mesh = plsc.VectorSubcoreMesh(core_axis_name="c", subcore_axis_name="s")

    @functools.partial(
        pl.kernel,
        mesh=mesh,
        out_type=jax.ShapeDtypeStruct((nrows, LATENT), jnp.float32),
        scratch_types=[
            pltpu.VMEM((hw,), jnp.int32),
            pltpu.VMEM((hw,), jnp.int32),
            pltpu.VMEM((hw, LATENT), jnp.float32),
            pltpu.VMEM((hw, LATENT), jnp.float32),
            pltpu.SemaphoreType.DMA,
            pltpu.SemaphoreType.DMA,
        ],
    )
    def gather_kernel(cb_hbm, idx_hbm, out_hbm, idx_a, idx_b, rows_a, rows_b,
                      sem_a, sem_b):
        wid = lax.axis_index("s") * nc + lax.axis_index("c")
        base = wid * bpw
        pltpu.sync_copy(idx_hbm.at[pl.ds(base, hw)], idx_a)
        pltpu.sync_copy(idx_hbm.at[pl.ds(base + hw, hw)], idx_b)
        ca = pltpu.async_copy(cb_hbm.at[idx_a], rows_a, sem_a)
        cb = pltpu.async_copy(cb_hbm.at[idx_b], rows_b, sem_b)
        ca.wait()
        pltpu.sync_copy(rows_a, out_hbm.at[pl.ds(base, hw)])
        cb.wait()
        pltpu.sync_copy(rows_b, out_hbm.at[pl.ds(base + hw, hw)])

    return gather_kernel(codebook, idx)


def kernel(x, enc_W, dec_W, codebook):
    z0, i0 = _encode_vq(x, enc_W, codebook, 0)
    z1, i1 = _encode_vq(x, enc_W, codebook, 1)
    zq0 = _sc_gather(codebook, i0.reshape(HALF_ROWS))
    zq1 = _sc_gather(codebook, i1.reshape(HALF_ROWS))
    d0, l0 = _decode(z0, zq0, dec_W, None, 0)
    decoded_images, l1 = _decode(z1, zq1, dec_W, d0, 1)
    codebook_indices = jnp.concatenate(
        [i0.reshape(HALF_B, N), i1.reshape(HALF_B, N)], axis=0)
    m = (l0[0, 0] + l1[0, 0]) / jnp.float32(ROWS * LATENT)
    q_loss = m + jnp.float32(BETA) * m
    return decoded_images, codebook_indices, q_loss


# speedup vs baseline: 1.4395x; 1.0914x over previous
import functools

import jax
import jax.numpy as jnp
from jax import lax
from jax.experimental import pallas as pl
from jax.experimental.pallas import tpu as pltpu
from jax.experimental.pallas import tpu_sc as plsc

B = 16
C = 3
H = W = 256
P = 16
NH = H // P
NW_ = W // P
N = NH * NW_
PATCH_DIM = C * P * P
LATENT = 256
K = 8192
BETA = 0.25
ROWS = B * N

HALF_B = B // 2
HALF_ROWS = HALF_B * N
ROW_TILE = 256
CODE_CHUNK = 2048
N_CHUNKS = K // CODE_CHUNK


def _enc_vq_body(x_ref, w_ref, cb_ref, z_ref, idx_ref, cn_ref):
    i = pl.program_id(0)

    @pl.when(i == 0)
    def _():
        for c in range(N_CHUNKS):
            cb = cb_ref[c * CODE_CHUNK:(c + 1) * CODE_CHUNK, :]
            cn_ref[0, c * CODE_CHUNK:(c + 1) * CODE_CHUNK] = jnp.sum(cb * cb, axis=1)

    xb = x_ref[0]
    x5 = xb.reshape(C, NH, P, NW_, P)
    pf = x5.transpose(1, 3, 0, 2, 4).reshape(N, PATCH_DIM)

    z = jnp.dot(pf, w_ref[...], preferred_element_type=jnp.float32)
    z_ref[...] = z
    zn = jnp.sum(z * z, axis=1, keepdims=True)
    best_v = None
    best_i = None
    for c in range(N_CHUNKS):
        cb = cb_ref[c * CODE_CHUNK:(c + 1) * CODE_CHUNK, :]
        cn = cn_ref[:, c * CODE_CHUNK:(c + 1) * CODE_CHUNK]
        mm = lax.dot_general(z, cb, (((1,), (1,)), ((), ())),
                             preferred_element_type=jnp.float32)
        d = (zn + cn) - 2.0 * mm
        vmin = jnp.min(d, axis=1, keepdims=True)
        col = lax.broadcasted_iota(jnp.int32, d.shape, 1) + (c * CODE_CHUNK)
        imin = jnp.min(jnp.where(d == vmin, col, jnp.int32(K)),
                       axis=1, keepdims=True)
        if best_v is None:
            best_v, best_i = vmin, imin
        else:
            take_new = vmin < best_v
            best_i = jnp.where(take_new, imin, best_i)
            best_v = jnp.where(take_new, vmin, best_v)
    idx_ref[...] = best_i[:, 0].reshape(1, 1, ROW_TILE)


def _encode_vq(x, enc_W, codebook, half):
    base = half * HALF_B
    return pl.pallas_call(
        _enc_vq_body,
        grid=(HALF_B,),
        in_specs=[
            pl.BlockSpec((1, C, H, W), lambda i, b=base: (i + b, 0, 0, 0)),
            pl.BlockSpec((PATCH_DIM, LATENT), lambda i: (0, 0)),
            pl.BlockSpec((K, LATENT), lambda i: (0, 0)),
        ],
        out_specs=[
            pl.BlockSpec((ROW_TILE, LATENT), lambda i: (i, 0)),
            pl.BlockSpec((1, 1, ROW_TILE), lambda i: (i, 0, 0)),
        ],
        out_shape=[
            jax.ShapeDtypeStruct((HALF_ROWS, LATENT), jnp.float32),
            jax.ShapeDtypeStruct((HALF_B, 1, ROW_TILE), jnp.int32),
        ],
        scratch_shapes=[pltpu.VMEM((1, K), jnp.float32)],
    )(x, enc_W, codebook)


def _dec_body(z_ref, zq_ref, w_ref, prev_ref, out_ref, loss_ref):
    del prev_ref
    i = pl.program_id(0)
    z = z_ref[...]
    zq = zq_ref[...]
    zq_st = z + (zq - z)
    dec = jnp.dot(zq_st, w_ref[...], preferred_element_type=jnp.float32)
    d5 = dec.reshape(NH, NW_, C, P, P)
    out_ref[...] = d5.transpose(2, 0, 3, 1, 4).reshape(1, C, H, W)
    dz = z - zq
    part = jnp.sum(dz * dz).reshape(1, 1)

    @pl.when(i == 0)
    def _():
        loss_ref[...] = part

    @pl.when(i > 0)
    def _():
        loss_ref[...] = loss_ref[...] + part


def _decode(z, zq, dec_W, prev, half):
    base = half * HALF_B
    in_specs = [
        pl.BlockSpec((ROW_TILE, LATENT), lambda i: (i, 0)),
        pl.BlockSpec((ROW_TILE, LATENT), lambda i: (i, 0)),
        pl.BlockSpec((LATENT, PATCH_DIM), lambda i: (0, 0)),
    ]
    args = [z, zq, dec_W]
    aliases = {}
    body = _dec_body
    if prev is not None:
        in_specs.append(pl.BlockSpec(memory_space=pltpu.MemorySpace.HBM))
        args.append(prev)
        aliases = {3: 0}
    else:
        def body(z_ref, zq_ref, w_ref, out_ref, loss_ref):
            return _dec_body(z_ref, zq_ref, w_ref, None, out_ref, loss_ref)
    return pl.pallas_call(
        body,
        grid=(HALF_B,),
        in_specs=in_specs,
        out_specs=[
            pl.BlockSpec((1, C, H, W), lambda i, b=base: (i + b, 0, 0, 0)),
            pl.BlockSpec((1, 1), lambda i: (0, 0)),
        ],
        out_shape=[
            jax.ShapeDtypeStruct((B, C, H, W), jnp.float32),
            jax.ShapeDtypeStruct((1, 1), jnp.float32),
        ],
        input_output_aliases=aliases,
    )(*args)


def _sc_gather(codebook, idx):
    nrows = idx.shape[0]
    info = plsc.get_sparse_core_info()
    nc, ns = info.num_cores, info.num_subcores
    nworkers = nc * ns
    bpw = nrows // nworkers
    hw = bpw // 2
    mesh = plsc.VectorSubcoreMesh(core_axis_name="c", subcore_axis_name="s")

    @functools.partial(
        pl.kernel,
        mesh=mesh,
        out_type=jax.ShapeDtypeStruct((nrows, LATENT), jnp.float32),
        scratch_types=[
            pltpu.VMEM((hw,), jnp.int32),
            pltpu.VMEM((hw,), jnp.int32),
            pltpu.VMEM((hw, LATENT), jnp.float32),
            pltpu.VMEM((hw, LATENT), jnp.float32),
            pltpu.SemaphoreType.DMA,
            pltpu.SemaphoreType.DMA,
        ],
    )
    def gather_kernel(cb_hbm, idx_hbm, out_hbm, idx_a, idx_b, rows_a, rows_b,
                      sem_a, sem_b):
        wid = lax.axis_index("s") * nc + lax.axis_index("c")
        base = wid * bpw
        pltpu.sync_copy(idx_hbm.at[pl.ds(base, hw)], idx_a)
        pltpu.sync_copy(idx_hbm.at[pl.ds(base + hw, hw)], idx_b)
        ca = pltpu.async_copy(cb_hbm.at[idx_a], rows_a, sem_a)
        cb = pltpu.async_copy(cb_hbm.at[idx_b], rows_b, sem_b)
        ca.wait()
        pltpu.sync_copy(rows_a, out_hbm.at[pl.ds(base, hw)])
        cb.wait()
        pltpu.sync_copy(rows_b, out_hbm.at[pl.ds(base + hw, hw)])

    return gather_kernel(codebook, idx)


def kernel(x, enc_W, dec_W, codebook):
    z0, i0 = _encode_vq(x, enc_W, codebook, 0)
    z1, i1 = _encode_vq(x, enc_W, codebook, 1)
    zq0 = _sc_gather(codebook, i0.reshape(HALF_ROWS))
    zq1 = _sc_gather(codebook, i1.reshape(HALF_ROWS))
    d0, l0 = _decode(z0, zq0, dec_W, None, 0)
    decoded_images, l1 = _decode(z1, zq1, dec_W, d0, 1)
    codebook_indices = jnp.concatenate(
        [i0.reshape(HALF_B, N), i1.reshape(HALF_B, N)], axis=0)
    m = (l0[0, 0] + l1[0, 0]) / jnp.float32(ROWS * LATENT)
    q_loss = m + jnp.float32(BETA) * m
    return decoded_images, codebook_indices, q_loss


# speedup vs baseline: 1.4461x; 1.0046x over previous
import functools

import jax
import jax.numpy as jnp
from jax import lax
from jax.experimental import pallas as pl
from jax.experimental.pallas import tpu as pltpu
from jax.experimental.pallas import tpu_sc as plsc

B = 16
C = 3
H = W = 256
P = 16
NH = H // P
NW_ = W // P
N = NH * NW_
PATCH_DIM = C * P * P
LATENT = 256
K = 8192
BETA = 0.25
ROWS = B * N

HALF_B = B // 2
HALF_ROWS = HALF_B * N
IMG_STEP = 1
ROW_TILE = IMG_STEP * N
CODE_CHUNK = 2048
N_CHUNKS = K // CODE_CHUNK


def _enc_vq_body(x_ref, w_ref, cb_ref, z_ref, idx_ref, cn_ref):
    i = pl.program_id(0)

    @pl.when(i == 0)
    def _():
        for c in range(N_CHUNKS):
            cb = cb_ref[c * CODE_CHUNK:(c + 1) * CODE_CHUNK, :]
            cn_ref[0, c * CODE_CHUNK:(c + 1) * CODE_CHUNK] = jnp.sum(cb * cb, axis=1)

    xb = x_ref[...]
    x6 = xb.reshape(IMG_STEP, C, NH, P, NW_, P)
    pf = x6.transpose(0, 2, 4, 1, 3, 5).reshape(ROW_TILE, PATCH_DIM)

    z = jnp.dot(pf, w_ref[...], preferred_element_type=jnp.float32)
    z_ref[...] = z
    zn = jnp.sum(z * z, axis=1, keepdims=True)
    best_v = None
    best_i = None
    for c in range(N_CHUNKS):
        cb = cb_ref[c * CODE_CHUNK:(c + 1) * CODE_CHUNK, :]
        cn = cn_ref[:, c * CODE_CHUNK:(c + 1) * CODE_CHUNK]
        mm = lax.dot_general(z, cb, (((1,), (1,)), ((), ())),
                             preferred_element_type=jnp.float32)
        d = (zn + cn) - 2.0 * mm
        vmin = jnp.min(d, axis=1, keepdims=True)
        col = lax.broadcasted_iota(jnp.int32, d.shape, 1) + (c * CODE_CHUNK)
        imin = jnp.min(jnp.where(d == vmin, col, jnp.int32(K)),
                       axis=1, keepdims=True)
        if best_v is None:
            best_v, best_i = vmin, imin
        else:
            take_new = vmin < best_v
            best_i = jnp.where(take_new, imin, best_i)
            best_v = jnp.where(take_new, vmin, best_v)
    idx_ref[...] = best_i[:, 0].reshape(1, 1, ROW_TILE)


def _encode_vq(x, enc_W, codebook, half):
    base = half * HALF_B
    return pl.pallas_call(
        _enc_vq_body,
        grid=(HALF_B // IMG_STEP,),
        in_specs=[
            pl.BlockSpec((IMG_STEP, C, H, W), lambda i, b=base // IMG_STEP: (i + b, 0, 0, 0)),
            pl.BlockSpec((PATCH_DIM, LATENT), lambda i: (0, 0)),
            pl.BlockSpec((K, LATENT), lambda i: (0, 0)),
        ],
        out_specs=[
            pl.BlockSpec((ROW_TILE, LATENT), lambda i: (i, 0)),
            pl.BlockSpec((1, 1, ROW_TILE), lambda i: (i, 0, 0)),
        ],
        out_shape=[
            jax.ShapeDtypeStruct((HALF_ROWS, LATENT), jnp.float32),
            jax.ShapeDtypeStruct((HALF_B // IMG_STEP, 1, ROW_TILE), jnp.int32),
        ],
        scratch_shapes=[pltpu.VMEM((1, K), jnp.float32)],
    )(x, enc_W, codebook)


def _dec_body(z_ref, zq_ref, w_ref, prev_ref, out_ref, loss_ref):
    del prev_ref
    i = pl.program_id(0)
    z = z_ref[...]
    zq = zq_ref[...]
    zq_st = z + (zq - z)
    dec = jnp.dot(zq_st, w_ref[...], preferred_element_type=jnp.float32)
    d6 = dec.reshape(IMG_STEP, NH, NW_, C, P, P)
    out_ref[...] = d6.transpose(0, 3, 1, 4, 2, 5).reshape(IMG_STEP, C, H, W)
    dz = z - zq
    part = jnp.sum(dz * dz).reshape(1, 1)

    @pl.when(i == 0)
    def _():
        loss_ref[...] = part

    @pl.when(i > 0)
    def _():
        loss_ref[...] = loss_ref[...] + part


def _decode(z, zq, dec_W, prev, half):
    base = half * HALF_B
    in_specs = [
        pl.BlockSpec((ROW_TILE, LATENT), lambda i: (i, 0)),
        pl.BlockSpec((ROW_TILE, LATENT), lambda i: (i, 0)),
        pl.BlockSpec((LATENT, PATCH_DIM), lambda i: (0, 0)),
    ]
    args = [z, zq, dec_W]
    aliases = {}
    body = _dec_body
    if prev is not None:
        in_specs.append(pl.BlockSpec(memory_space=pltpu.MemorySpace.HBM))
        args.append(prev)
        aliases = {3: 0}
    else:
        def body(z_ref, zq_ref, w_ref, out_ref, loss_ref):
            return _dec_body(z_ref, zq_ref, w_ref, None, out_ref, loss_ref)
    return pl.pallas_call(
        body,
        grid=(HALF_B // IMG_STEP,),
        in_specs=in_specs,
        out_specs=[
            pl.BlockSpec((IMG_STEP, C, H, W), lambda i, b=base // IMG_STEP: (i + b, 0, 0, 0)),
            pl.BlockSpec((1, 1), lambda i: (0, 0)),
        ],
        out_shape=[
            jax.ShapeDtypeStruct((B, C, H, W), jnp.float32),
            jax.ShapeDtypeStruct((1, 1), jnp.float32),
        ],
        input_output_aliases=aliases,
    )(*args)


def _sc_gather(codebook, idx):
    nrows = idx.shape[0]
    info = plsc.get_sparse_core_info()
    nc, ns = info.num_cores, info.num_subcores
    nworkers = nc * ns
    bpw = nrows // nworkers
    hw = bpw // 2
    mesh = plsc.VectorSubcoreMesh(core_axis_name="c", subcore_axis_name="s")

    @functools.partial(
        pl.kernel,
        mesh=mesh,
        out_type=jax.ShapeDtypeStruct((nrows, LATENT), jnp.float32),
        scratch_types=[
            pltpu.VMEM((hw,), jnp.int32),
            pltpu.VMEM((hw,), jnp.int32),
            pltpu.VMEM((hw, LATENT), jnp.float32),
            pltpu.VMEM((hw, LATENT), jnp.float32),
            pltpu.SemaphoreType.DMA,
            pltpu.SemaphoreType.DMA,
        ],
    )
    def gather_kernel(cb_hbm, idx_hbm, out_hbm, idx_a, idx_b, rows_a, rows_b,
                      sem_a, sem_b):
        wid = lax.axis_index("s") * nc + lax.axis_index("c")
        base = wid * bpw
        pltpu.sync_copy(idx_hbm.at[pl.ds(base, hw)], idx_a)
        pltpu.sync_copy(idx_hbm.at[pl.ds(base + hw, hw)], idx_b)
        ca = pltpu.async_copy(cb_hbm.at[idx_a], rows_a, sem_a)
        cb = pltpu.async_copy(cb_hbm.at[idx_b], rows_b, sem_b)
        ca.wait()
        pltpu.sync_copy(rows_a, out_hbm.at[pl.ds(base, hw)])
        cb.wait()
        pltpu.sync_copy(rows_b, out_hbm.at[pl.ds(base + hw, hw)])

    return gather_kernel(codebook, idx)


def kernel(x, enc_W, dec_W, codebook):
    z0, i0 = _encode_vq(x, enc_W, codebook, 0)
    z1, i1 = _encode_vq(x, enc_W, codebook, 1)
    zq0 = _sc_gather(codebook, i0.reshape(HALF_ROWS))
    zq1 = _sc_gather(codebook, i1.reshape(HALF_ROWS))
    d0, l0 = _decode(z0, zq0, dec_W, None, 0)
    decoded_images, l1 = _decode(z1, zq1, dec_W, d0, 1)
    codebook_indices = jnp.concatenate(
        [i0.reshape(HALF_B, N), i1.reshape(HALF_B, N)], axis=0)
    m = (l0[0, 0] + l1[0, 0]) / jnp.float32(ROWS * LATENT)
    q_loss = m + jnp.float32(BETA) * m
    return decoded_images, codebook_indices, q_loss
